# baseline (speedup 1.0000x reference)
import functools

import jax
import jax.numpy as jnp
from jax import lax
from jax.experimental import pallas as pl
from jax.experimental.pallas import tpu as pltpu
from jax.experimental.pallas import tpu_sc as plsc

N = 10000
F = 128
E = 320000
C = 10

NPAD = 10240
TRASH = N
NC = 2
NS = 16
NW = NC * NS
EPW = 10240
EPAD = NW * EPW
CHUNK = 64
NCH = EPW // CHUNK
NBUF = 4
NPH = 2
DCHUNK = 128
DNCH = EPW // DCHUNK
RPW = NPAD // NS

@functools.lru_cache(maxsize=None)
def _mesh():
    return plsc.VectorSubcoreMesh(core_axis_name="c", subcore_axis_name="s",
                                  num_cores=NC, num_subcores=NS)


def _deg_body(dst_hbm, out_hbm, idx_v, ones_v, zeros_v, deg_sh):
    cid = lax.axis_index("c")
    sid = lax.axis_index("s")
    wid = cid * NS + sid

    def fill_ones(i, _):
        ones_v[pl.ds(i * 16, 16)] = jnp.full((16,), 1.0, jnp.float32)
        return 0

    lax.fori_loop(0, DCHUNK // 16, fill_ones, 0)

    def fill_zeros(i, _):
        zeros_v[pl.ds(i * 16, 16)] = jnp.zeros((16,), jnp.float32)
        return 0

    lax.fori_loop(0, RPW // 16, fill_zeros, 0)

    pltpu.sync_copy(zeros_v, deg_sh.at[pl.ds(sid * RPW, RPW)])
    pltpu.sync_copy(dst_hbm.at[wid], idx_v)
    plsc.subcore_barrier()

    def scatter(j, _):
        pltpu.sync_copy(ones_v, deg_sh.at[idx_v.at[j]], add=True)
        return 0

    lax.fori_loop(0, DNCH, scatter, 0)
    plsc.subcore_barrier()
    pltpu.sync_copy(deg_sh.at[pl.ds(sid * RPW, RPW)],
                    out_hbm.at[cid, pl.ds(sid * RPW, RPW)])


@functools.lru_cache(maxsize=None)
def _deg_call():
    return pl.kernel(
        _deg_body,
        out_type=jax.ShapeDtypeStruct((NC, NPAD), jnp.float32),
        mesh=_mesh(),
        scratch_types=[
            pltpu.VMEM((DNCH, DCHUNK), jnp.int32),
            pltpu.VMEM((DCHUNK,), jnp.float32),
            pltpu.VMEM((RPW,), jnp.float32),
            pltpu.VMEM_SHARED((NPAD,), jnp.float32),
        ],
    )


def _msg_body(hs_hbm, src_hbm, dst_hbm, out_hbm, src_v, dst_v, rows_v,
              acc_sh, sem):
    cid = lax.axis_index("c")
    sid = lax.axis_index("s")
    wid = cid * NS + sid

    def fill_zeros(i, _):
        rows_v[0, i // 8, pl.ds((i % 8) * 16, 16)] = jnp.zeros((16,),
                                                               jnp.float32)
        return 0

    lax.fori_loop(0, CHUNK * 8, fill_zeros, 0)

    def zero_acc(i, _):
        pltpu.sync_copy(rows_v.at[0],
                        acc_sh.at[pl.ds(sid * RPW + i * CHUNK, CHUNK)])
        return 0

    lax.fori_loop(0, RPW // CHUNK, zero_acc, 0)

    plsc.subcore_barrier()

    KPH = NCH // NPH

    for ph in range(NPH):
        pltpu.sync_copy(src_hbm.at[wid, pl.ds(ph * KPH * CHUNK, KPH * CHUNK)],
                        src_v)
        pltpu.sync_copy(dst_hbm.at[wid, pl.ds(ph * KPH * CHUNK, KPH * CHUNK)],
                        dst_v)

        def issue(k, buf):
            pltpu.async_copy(hs_hbm.at[src_v.at[pl.ds(k * CHUNK, CHUNK)]],
                             rows_v.at[buf], sem.at[buf])

        for j in range(NBUF):
            issue(j, j)

        def group(g, _):
            base = g * NBUF
            for j in range(NBUF):
                pltpu.make_async_copy(
                    hs_hbm.at[src_v.at[pl.ds((base + j) * CHUNK, CHUNK)]],
                    rows_v.at[j], sem.at[j]).wait()
                pltpu.sync_copy(
                    rows_v.at[j],
                    acc_sh.at[dst_v.at[pl.ds((base + j) * CHUNK, CHUNK)]],
                    add=True)

                @pl.when(base + NBUF + j < KPH)
                def _():
                    issue(base + NBUF + j, j)

            return 0

        lax.fori_loop(0, KPH // NBUF, group, 0)

    plsc.subcore_barrier()
    pltpu.sync_copy(acc_sh.at[pl.ds(sid * RPW, RPW)],
                    out_hbm.at[cid, pl.ds(sid * RPW, RPW)])


@functools.lru_cache(maxsize=None)
def _msg_call():
    return pl.kernel(
        _msg_body,
        out_type=jax.ShapeDtypeStruct((NC, NPAD, F), jnp.float32),
        mesh=_mesh(),
        scratch_types=[
            pltpu.VMEM((EPW // NPH,), jnp.int32),
            pltpu.VMEM((EPW // NPH,), jnp.int32),
            pltpu.VMEM((NBUF, CHUNK, F), jnp.float32),
            pltpu.VMEM_SHARED((NPAD, F), jnp.float32),
            pltpu.SemaphoreType.DMA((NBUF,)),
        ],
    )


def _gru_h1_body(x_ref, w0a, wihTa, whhTa, biha, bhha,
                 w0b, wihTb, whhTb, bihb, bhhb,
                 h1_ref, w2_ref):
    def evolve(w0, wihT, whhT, bih, bhh):
        gi = jnp.dot(w0[...], wihT[...], preferred_element_type=jnp.float32)
        gi = gi + bih[...]
        gh = jnp.dot(w0[...], whhT[...], preferred_element_type=jnp.float32)
        gh = gh + bhh[...]
        r = jax.nn.sigmoid(gi[:, 0:F] + gh[:, 0:F])
        z = jax.nn.sigmoid(gi[:, F:2 * F] + gh[:, F:2 * F])
        n = jnp.tanh(gi[:, 2 * F:3 * F] + r * gh[:, 2 * F:3 * F])
        return (1.0 - z) * n + z * w0[...]

    w1 = evolve(w0a, wihTa, whhTa, biha, bhha)
    w2_ref[...] = evolve(w0b, wihTb, whhTb, bihb, bhhb)
    h1_ref[...] = jnp.dot(x_ref[...], w1, preferred_element_type=jnp.float32)


def _scale_body(deg_ref, h1_ref, hs_ref, dinv_ref):
    deg = deg_ref[0] + deg_ref[1] + 1.0
    dinv = lax.rsqrt(deg)
    dinv_ref[...] = dinv
    hs_ref[...] = h1_ref[...] * dinv[:, :, None]


def _mid_body(acc_ref, hs_ref, dinv_ref, w2_ref, out_ref):
    h = (acc_ref[0] + acc_ref[1] + hs_ref[...]) * dinv_ref[...]
    h = jnp.maximum(h, 0.0)
    h2 = jnp.dot(h, w2_ref[...], preferred_element_type=jnp.float32)
    out_ref[...] = h2 * dinv_ref[...]


def _final_body(acc_ref, hs_ref, dinv_ref, wlinT_ref, blin_ref, out_ref):
    h = (acc_ref[0] + acc_ref[1] + hs_ref[...]) * dinv_ref[...]
    h = jnp.maximum(h, 0.0)
    logits = jnp.dot(h, wlinT_ref[...], preferred_element_type=jnp.float32)
    logits = logits + blin_ref[...]
    col = lax.broadcasted_iota(jnp.int32, logits.shape, 1)
    l = jnp.where(col < C, logits, -1e30)
    m = jnp.max(l, axis=1, keepdims=True)
    ex = jnp.exp(l - m)
    lse = jnp.log(jnp.sum(ex, axis=1, keepdims=True))
    out_ref[...] = l - m - lse


_RB = 1024
_GRID = NPAD // _RB


def _spec(shape, row_dim=None):
    if row_dim is None:
        return pl.BlockSpec(shape, lambda i: (0,) * len(shape))
    blk = tuple(_RB if d == row_dim else s for d, s in enumerate(shape))
    idx = lambda i: tuple(i if d == row_dim else 0 for d in range(len(shape)))
    return pl.BlockSpec(blk, idx)


def _tc_rows_call(body, specs, out_shape):
    return pl.pallas_call(
        body,
        grid=(_GRID,),
        in_specs=[_spec(s, r) for s, r in specs],
        out_specs=_spec(out_shape, 0),
        out_shape=jax.ShapeDtypeStruct(out_shape, jnp.float32),
    )


def kernel(x, edge_index, w0_1, wih_1, whh_1, bih_1, bhh_1,
           w0_2, wih_2, whh_2, bih_2, bhh_2, w_lin, b_lin):
    src = edge_index[0]
    dst = edge_index[1]
    pad = EPAD - E
    trash = TRASH + jnp.arange(pad, dtype=jnp.int32) % (NPAD - N)
    src_pad = jnp.arange(pad, dtype=jnp.int32) % N
    src_p = jnp.concatenate([src, src_pad])
    dst_p = jnp.concatenate([dst, trash])
    src_r = src_p.reshape(NW, EPW)
    dst_r = dst_p.reshape(NW, EPW)
    dst_d = dst_p.reshape(NW, DNCH, DCHUNK)
    x_pad = jnp.pad(x, ((0, NPAD - N), (0, 0)))
    wlinT = jnp.zeros((F, F), jnp.float32).at[:C].set(w_lin).T
    blin = jnp.zeros((1, F), jnp.float32).at[0, :C].set(b_lin)

    deg2 = _deg_call()(dst_d)

    gru_in = (
        x_pad,
        w0_1, wih_1.T, whh_1.T, bih_1.reshape(1, 3 * F), bhh_1.reshape(1, 3 * F),
        w0_2, wih_2.T, whh_2.T, bih_2.reshape(1, 3 * F), bhh_2.reshape(1, 3 * F),
    )
    h1, w2 = pl.pallas_call(
        _gru_h1_body,
        out_shape=[
            jax.ShapeDtypeStruct((NPAD, F), jnp.float32),
            jax.ShapeDtypeStruct((F, F), jnp.float32),
        ],
    )(*gru_in)

    hs1g, dinv_t = pl.pallas_call(
        _scale_body,
        out_shape=[
            jax.ShapeDtypeStruct((NPAD // F, F, F), jnp.float32),
            jax.ShapeDtypeStruct((NPAD // F, F), jnp.float32),
        ],
    )(deg2.reshape(NC, NPAD // F, F), h1.reshape(NPAD // F, F, F))
    hs1 = hs1g.reshape(NPAD, F)
    dinv = dinv_t.reshape(NPAD, 1)

    acc1 = _msg_call()(hs1, src_r, dst_r)

    hs2 = _tc_rows_call(
        _mid_body,
        [((NC, NPAD, F), 1), ((NPAD, F), 0), ((NPAD, 1), 0), ((F, F), None)],
        (NPAD, F),
    )(acc1, hs1, dinv, w2)
    acc2 = _msg_call()(hs2, src_r, dst_r)

    out = _tc_rows_call(
        _final_body,
        [((NC, NPAD, F), 1), ((NPAD, F), 0), ((NPAD, 1), 0), ((F, F), None),
         ((1, F), None)],
        (NPAD, F),
    )(acc2, hs2, dinv, wlinT, blin)
    return out[:N, :C]


# speedup vs baseline: 1.0033x; 1.0033x over previous
import functools

import jax
import jax.numpy as jnp
from jax import lax
from jax.experimental import pallas as pl
from jax.experimental.pallas import tpu as pltpu
from jax.experimental.pallas import tpu_sc as plsc

N = 10000
F = 128
E = 320000
C = 10

NPAD = 10240
TRASH = N
NC = 2
NS = 16
NW = NC * NS
EPW = 10240
EPAD = NW * EPW
CHUNK = 32
NCH = EPW // CHUNK
NBUF = 8
NPH = 2
DCHUNK = 128
DNCH = EPW // DCHUNK
RPW = NPAD // NS

@functools.lru_cache(maxsize=None)
def _mesh():
    return plsc.VectorSubcoreMesh(core_axis_name="c", subcore_axis_name="s",
                                  num_cores=NC, num_subcores=NS)


def _deg_body(dst_hbm, out_hbm, idx_v, ones_v, zeros_v, deg_sh):
    cid = lax.axis_index("c")
    sid = lax.axis_index("s")
    wid = cid * NS + sid

    def fill_ones(i, _):
        ones_v[pl.ds(i * 16, 16)] = jnp.full((16,), 1.0, jnp.float32)
        return 0

    lax.fori_loop(0, DCHUNK // 16, fill_ones, 0)

    def fill_zeros(i, _):
        zeros_v[pl.ds(i * 16, 16)] = jnp.zeros((16,), jnp.float32)
        return 0

    lax.fori_loop(0, RPW // 16, fill_zeros, 0)

    pltpu.sync_copy(zeros_v, deg_sh.at[pl.ds(sid * RPW, RPW)])
    pltpu.sync_copy(dst_hbm.at[wid], idx_v)
    plsc.subcore_barrier()

    def scatter(j, _):
        pltpu.sync_copy(ones_v, deg_sh.at[idx_v.at[j]], add=True)
        return 0

    lax.fori_loop(0, DNCH, scatter, 0)
    plsc.subcore_barrier()
    pltpu.sync_copy(deg_sh.at[pl.ds(sid * RPW, RPW)],
                    out_hbm.at[cid, pl.ds(sid * RPW, RPW)])


@functools.lru_cache(maxsize=None)
def _deg_call():
    return pl.kernel(
        _deg_body,
        out_type=jax.ShapeDtypeStruct((NC, NPAD), jnp.float32),
        mesh=_mesh(),
        scratch_types=[
            pltpu.VMEM((DNCH, DCHUNK), jnp.int32),
            pltpu.VMEM((DCHUNK,), jnp.float32),
            pltpu.VMEM((RPW,), jnp.float32),
            pltpu.VMEM_SHARED((NPAD,), jnp.float32),
        ],
    )


def _msg_body(hs_hbm, src_hbm, dst_hbm, out_hbm, src_v, dst_v, rows_v,
              acc_sh, sem):
    cid = lax.axis_index("c")
    sid = lax.axis_index("s")
    wid = cid * NS + sid

    def fill_zeros(i, _):
        rows_v[0, i // 8, pl.ds((i % 8) * 16, 16)] = jnp.zeros((16,),
                                                               jnp.float32)
        return 0

    lax.fori_loop(0, CHUNK * 8, fill_zeros, 0)

    def zero_acc(i, _):
        pltpu.sync_copy(rows_v.at[0],
                        acc_sh.at[pl.ds(sid * RPW + i * CHUNK, CHUNK)])
        return 0

    lax.fori_loop(0, RPW // CHUNK, zero_acc, 0)

    plsc.subcore_barrier()

    KPH = NCH // NPH

    for ph in range(NPH):
        pltpu.sync_copy(src_hbm.at[wid, pl.ds(ph * KPH * CHUNK, KPH * CHUNK)],
                        src_v)
        pltpu.sync_copy(dst_hbm.at[wid, pl.ds(ph * KPH * CHUNK, KPH * CHUNK)],
                        dst_v)

        def issue(k, buf):
            pltpu.async_copy(hs_hbm.at[src_v.at[pl.ds(k * CHUNK, CHUNK)]],
                             rows_v.at[buf], sem.at[buf])

        for j in range(NBUF):
            issue(j, j)

        def group(g, _):
            base = g * NBUF
            for j in range(NBUF):
                pltpu.make_async_copy(
                    hs_hbm.at[src_v.at[pl.ds((base + j) * CHUNK, CHUNK)]],
                    rows_v.at[j], sem.at[j]).wait()
                pltpu.sync_copy(
                    rows_v.at[j],
                    acc_sh.at[dst_v.at[pl.ds((base + j) * CHUNK, CHUNK)]],
                    add=True)

                @pl.when(base + NBUF + j < KPH)
                def _():
                    issue(base + NBUF + j, j)

            return 0

        lax.fori_loop(0, KPH // NBUF, group, 0)

    plsc.subcore_barrier()
    pltpu.sync_copy(acc_sh.at[pl.ds(sid * RPW, RPW)],
                    out_hbm.at[cid, pl.ds(sid * RPW, RPW)])


@functools.lru_cache(maxsize=None)
def _msg_call():
    return pl.kernel(
        _msg_body,
        out_type=jax.ShapeDtypeStruct((NC, NPAD, F), jnp.float32),
        mesh=_mesh(),
        scratch_types=[
            pltpu.VMEM((EPW // NPH,), jnp.int32),
            pltpu.VMEM((EPW // NPH,), jnp.int32),
            pltpu.VMEM((NBUF, CHUNK, F), jnp.float32),
            pltpu.VMEM_SHARED((NPAD, F), jnp.float32),
            pltpu.SemaphoreType.DMA((NBUF,)),
        ],
    )


def _gru_h1_body(x_ref, w0a, wihTa, whhTa, biha, bhha,
                 w0b, wihTb, whhTb, bihb, bhhb,
                 h1_ref, w2_ref):
    def evolve(w0, wihT, whhT, bih, bhh):
        gi = jnp.dot(w0[...], wihT[...], preferred_element_type=jnp.float32)
        gi = gi + bih[...]
        gh = jnp.dot(w0[...], whhT[...], preferred_element_type=jnp.float32)
        gh = gh + bhh[...]
        r = jax.nn.sigmoid(gi[:, 0:F] + gh[:, 0:F])
        z = jax.nn.sigmoid(gi[:, F:2 * F] + gh[:, F:2 * F])
        n = jnp.tanh(gi[:, 2 * F:3 * F] + r * gh[:, 2 * F:3 * F])
        return (1.0 - z) * n + z * w0[...]

    w1 = evolve(w0a, wihTa, whhTa, biha, bhha)
    w2_ref[...] = evolve(w0b, wihTb, whhTb, bihb, bhhb)
    h1_ref[...] = jnp.dot(x_ref[...], w1, preferred_element_type=jnp.float32)


def _scale_body(deg_ref, h1_ref, hs_ref, dinv_ref):
    deg = deg_ref[0] + deg_ref[1] + 1.0
    dinv = lax.rsqrt(deg)
    dinv_ref[...] = dinv
    hs_ref[...] = h1_ref[...] * dinv[:, :, None]


def _mid_body(acc_ref, hs_ref, dinv_ref, w2_ref, out_ref):
    h = (acc_ref[0] + acc_ref[1] + hs_ref[...]) * dinv_ref[...]
    h = jnp.maximum(h, 0.0)
    h2 = jnp.dot(h, w2_ref[...], preferred_element_type=jnp.float32)
    out_ref[...] = h2 * dinv_ref[...]


def _final_body(acc_ref, hs_ref, dinv_ref, wlinT_ref, blin_ref, out_ref):
    h = (acc_ref[0] + acc_ref[1] + hs_ref[...]) * dinv_ref[...]
    h = jnp.maximum(h, 0.0)
    logits = jnp.dot(h, wlinT_ref[...], preferred_element_type=jnp.float32)
    logits = logits + blin_ref[...]
    col = lax.broadcasted_iota(jnp.int32, logits.shape, 1)
    l = jnp.where(col < C, logits, -1e30)
    m = jnp.max(l, axis=1, keepdims=True)
    ex = jnp.exp(l - m)
    lse = jnp.log(jnp.sum(ex, axis=1, keepdims=True))
    out_ref[...] = l - m - lse


_RB = 1024
_GRID = NPAD // _RB


def _spec(shape, row_dim=None):
    if row_dim is None:
        return pl.BlockSpec(shape, lambda i: (0,) * len(shape))
    blk = tuple(_RB if d == row_dim else s for d, s in enumerate(shape))
    idx = lambda i: tuple(i if d == row_dim else 0 for d in range(len(shape)))
    return pl.BlockSpec(blk, idx)


def _tc_rows_call(body, specs, out_shape):
    return pl.pallas_call(
        body,
        grid=(_GRID,),
        in_specs=[_spec(s, r) for s, r in specs],
        out_specs=_spec(out_shape, 0),
        out_shape=jax.ShapeDtypeStruct(out_shape, jnp.float32),
    )


def kernel(x, edge_index, w0_1, wih_1, whh_1, bih_1, bhh_1,
           w0_2, wih_2, whh_2, bih_2, bhh_2, w_lin, b_lin):
    src = edge_index[0]
    dst = edge_index[1]
    pad = EPAD - E
    trash = TRASH + jnp.arange(pad, dtype=jnp.int32) % (NPAD - N)
    src_pad = jnp.arange(pad, dtype=jnp.int32) % N
    src_p = jnp.concatenate([src, src_pad])
    dst_p = jnp.concatenate([dst, trash])
    src_r = src_p.reshape(NW, EPW)
    dst_r = dst_p.reshape(NW, EPW)
    dst_d = dst_p.reshape(NW, DNCH, DCHUNK)
    x_pad = jnp.pad(x, ((0, NPAD - N), (0, 0)))
    wlinT = jnp.zeros((F, F), jnp.float32).at[:C].set(w_lin).T
    blin = jnp.zeros((1, F), jnp.float32).at[0, :C].set(b_lin)

    deg2 = _deg_call()(dst_d)

    gru_in = (
        x_pad,
        w0_1, wih_1.T, whh_1.T, bih_1.reshape(1, 3 * F), bhh_1.reshape(1, 3 * F),
        w0_2, wih_2.T, whh_2.T, bih_2.reshape(1, 3 * F), bhh_2.reshape(1, 3 * F),
    )
    h1, w2 = pl.pallas_call(
        _gru_h1_body,
        out_shape=[
            jax.ShapeDtypeStruct((NPAD, F), jnp.float32),
            jax.ShapeDtypeStruct((F, F), jnp.float32),
        ],
    )(*gru_in)

    hs1g, dinv_t = pl.pallas_call(
        _scale_body,
        out_shape=[
            jax.ShapeDtypeStruct((NPAD // F, F, F), jnp.float32),
            jax.ShapeDtypeStruct((NPAD // F, F), jnp.float32),
        ],
    )(deg2.reshape(NC, NPAD // F, F), h1.reshape(NPAD // F, F, F))
    hs1 = hs1g.reshape(NPAD, F)
    dinv = dinv_t.reshape(NPAD, 1)

    acc1 = _msg_call()(hs1, src_r, dst_r)

    hs2 = _tc_rows_call(
        _mid_body,
        [((NC, NPAD, F), 1), ((NPAD, F), 0), ((NPAD, 1), 0), ((F, F), None)],
        (NPAD, F),
    )(acc1, hs1, dinv, w2)
    acc2 = _msg_call()(hs2, src_r, dst_r)

    out = _tc_rows_call(
        _final_body,
        [((NC, NPAD, F), 1), ((NPAD, F), 0), ((NPAD, 1), 0), ((F, F), None),
         ((1, F), None)],
        (NPAD, F),
    )(acc2, hs2, dinv, wlinT, blin)
    return out[:N, :C]
